# baseline (speedup 1.0000x reference)
import jax
import jax.numpy as jnp
from jax.experimental import pallas as pl
from jax.experimental.pallas import tpu as pltpu

NUM_Q = 100000
D = 256
BLOCK = 4000
NBLK = NUM_Q // BLOCK


def _copy_body(e_in, p_in, e_out, p_out, se, sp, lsem, ssem):
    srcs = (e_in, p_in)
    dsts = (e_out, p_out)
    scr = (se, sp)

    def load(i):
        slot = i % 2
        cs = []
        for t in range(2):
            c = pltpu.make_async_copy(
                srcs[t].at[pl.ds(i * BLOCK, BLOCK)], scr[t].at[slot],
                lsem.at[t, slot])
            c.start()
            cs.append(c)
        return cs

    def store(i):
        slot = i % 2
        cs = []
        for t in range(2):
            c = pltpu.make_async_copy(
                scr[t].at[slot], dsts[t].at[pl.ds(i * BLOCK, BLOCK)],
                ssem.at[t, slot])
            c.start()
            cs.append(c)
        return cs

    loads = [None] * NBLK
    stores = [None] * NBLK
    loads[0] = load(0)
    for i in range(NBLK):
        if i + 1 < NBLK:
            if i - 1 >= 0:
                for c in stores[i - 1]:
                    c.wait()
            loads[i + 1] = load(i + 1)
        for c in loads[i]:
            c.wait()
        stores[i] = store(i)
    for c in stores[NBLK - 2]:
        c.wait()
    for c in stores[NBLK - 1]:
        c.wait()


def kernel(batch_size, query_embed_weight, query_pos_weight):
    out = jax.ShapeDtypeStruct((NUM_Q, D), jnp.float32)
    query_embed, query_pos = pl.pallas_call(
        _copy_body,
        in_specs=[
            pl.BlockSpec(memory_space=pl.ANY),
            pl.BlockSpec(memory_space=pl.ANY),
        ],
        out_specs=[
            pl.BlockSpec(memory_space=pl.ANY),
            pl.BlockSpec(memory_space=pl.ANY),
        ],
        out_shape=[out, out],
        scratch_shapes=[
            pltpu.VMEM((2, BLOCK, D), jnp.float32),
            pltpu.VMEM((2, BLOCK, D), jnp.float32),
            pltpu.SemaphoreType.DMA((2, 2)),
            pltpu.SemaphoreType.DMA((2, 2)),
        ],
    )(query_embed_weight, query_pos_weight)
    return (query_embed, query_pos)


# speedup vs baseline: 1.0257x; 1.0257x over previous
import jax
import jax.numpy as jnp
from jax.experimental import pallas as pl
from jax.experimental.pallas import tpu as pltpu

NUM_Q = 100000
D = 256
BLOCK = 2000
NBLK = NUM_Q // BLOCK
SLOTS = 4


def _copy_body(e_in, p_in, e_out, p_out, se, sp, lsem, ssem):
    srcs = (e_in, p_in)
    dsts = (e_out, p_out)
    scr = (se, sp)

    def load(i):
        slot = i % SLOTS
        cs = []
        for t in range(2):
            c = pltpu.make_async_copy(
                srcs[t].at[pl.ds(i * BLOCK, BLOCK)], scr[t].at[slot],
                lsem.at[t, slot])
            c.start()
            cs.append(c)
        return cs

    def store(i):
        slot = i % SLOTS
        cs = []
        for t in range(2):
            c = pltpu.make_async_copy(
                scr[t].at[slot], dsts[t].at[pl.ds(i * BLOCK, BLOCK)],
                ssem.at[t, slot])
            c.start()
            cs.append(c)
        return cs

    loads = [None] * NBLK
    stores = [None] * NBLK
    loads[0] = load(0)
    for i in range(NBLK):
        if i + 1 < NBLK:
            if i - (SLOTS - 1) >= 0:
                for c in stores[i - (SLOTS - 1)]:
                    c.wait()
            loads[i + 1] = load(i + 1)
        for c in loads[i]:
            c.wait()
        stores[i] = store(i)
    for j in range(max(0, NBLK - SLOTS), NBLK):
        for c in stores[j]:
            c.wait()


def kernel(batch_size, query_embed_weight, query_pos_weight):
    out = jax.ShapeDtypeStruct((NUM_Q, D), jnp.float32)
    query_embed, query_pos = pl.pallas_call(
        _copy_body,
        in_specs=[
            pl.BlockSpec(memory_space=pl.ANY),
            pl.BlockSpec(memory_space=pl.ANY),
        ],
        out_specs=[
            pl.BlockSpec(memory_space=pl.ANY),
            pl.BlockSpec(memory_space=pl.ANY),
        ],
        out_shape=[out, out],
        scratch_shapes=[
            pltpu.VMEM((SLOTS, BLOCK, D), jnp.float32),
            pltpu.VMEM((SLOTS, BLOCK, D), jnp.float32),
            pltpu.SemaphoreType.DMA((2, SLOTS)),
            pltpu.SemaphoreType.DMA((2, SLOTS)),
        ],
    )(query_embed_weight, query_pos_weight)
    return (query_embed, query_pos)
